# baseline (speedup 1.0000x reference)
import functools

import jax
import jax.numpy as jnp
from jax import lax
from jax.experimental import pallas as pl
from jax.experimental.pallas import tpu as pltpu

_NCPAD = 128


def _dot(a, b):
    return lax.dot_general(a, b, (((1,), (0,)), ((), ())),
                           preferred_element_type=jnp.float32)


def _xw_body(x_ref, w_ref, o_ref):
    o_ref[...] = _dot(x_ref[...].astype(jnp.bfloat16),
                      w_ref[...]).astype(jnp.bfloat16)


def _pass1_body(adj_ref, xw_ref, b1_ref, w4_ref, g_ref):
    a = adj_ref[...].astype(jnp.bfloat16)
    h = _dot(a, xw_ref[...]) + b1_ref[...]
    h = jnp.maximum(h, 0.0).astype(jnp.bfloat16)
    g_ref[...] = _dot(h, w4_ref[...]).astype(jnp.bfloat16)


def _pass2_body(nclass, adj_ref, g_ref, b4_ref, o_ref):
    a = adj_ref[...].astype(jnp.bfloat16)
    z = _dot(a, g_ref[...]) + b4_ref[...]
    col = lax.broadcasted_iota(jnp.int32, z.shape, 1)
    zm = jnp.where(col < nclass, z, -jnp.inf)
    m = jnp.max(zm, axis=1, keepdims=True)
    e = jnp.exp(zm - m)
    lse = jnp.log(jnp.sum(e, axis=1, keepdims=True))
    o_ref[...] = (z - m) - lse


def _pick_block(n):
    for b in (200, 128, 100, 64, 50, 40, 32, 25, 20, 16, 10, 8, 5, 4, 2):
        if n % b == 0:
            return b
    return n


@jax.jit
def kernel(x, adj, W1, b1, W4, b4):
    n, nfeat = x.shape
    nhid = W1.shape[1]
    nclass = W4.shape[1]

    w1b = W1.astype(jnp.bfloat16)
    w4b = jnp.pad(W4, ((0, 0), (0, _NCPAD - nclass))).astype(jnp.bfloat16)
    b1r = b1.reshape(1, nhid)
    b4r = jnp.pad(b4, (0, _NCPAD - nclass)).reshape(1, _NCPAD)

    bx = _pick_block(n) * 10 if n % (_pick_block(n) * 10) == 0 else _pick_block(n)
    xw1 = pl.pallas_call(
        _xw_body,
        grid=(n // bx,),
        in_specs=[
            pl.BlockSpec((bx, nfeat), lambda i: (i, 0)),
            pl.BlockSpec((nfeat, nhid), lambda i: (0, 0)),
        ],
        out_specs=pl.BlockSpec((bx, nhid), lambda i: (i, 0)),
        out_shape=jax.ShapeDtypeStruct((n, nhid), jnp.bfloat16),
        compiler_params=pltpu.CompilerParams(
            dimension_semantics=("parallel",)),
    )(x, w1b)

    bi = _pick_block(n)
    g = pl.pallas_call(
        _pass1_body,
        grid=(n // bi,),
        in_specs=[
            pl.BlockSpec((bi, n), lambda i: (i, 0)),
            pl.BlockSpec((n, nhid), lambda i: (0, 0)),
            pl.BlockSpec((1, nhid), lambda i: (0, 0)),
            pl.BlockSpec((nhid, _NCPAD), lambda i: (0, 0)),
        ],
        out_specs=pl.BlockSpec((bi, _NCPAD), lambda i: (i, 0)),
        out_shape=jax.ShapeDtypeStruct((n, _NCPAD), jnp.bfloat16),
        compiler_params=pltpu.CompilerParams(
            dimension_semantics=("parallel",)),
    )(adj, xw1, b1r, w4b)

    out = pl.pallas_call(
        functools.partial(_pass2_body, nclass),
        grid=(n // bi,),
        in_specs=[
            pl.BlockSpec((bi, n), lambda i: (i, 0)),
            pl.BlockSpec((n, _NCPAD), lambda i: (0, 0)),
            pl.BlockSpec((1, _NCPAD), lambda i: (0, 0)),
        ],
        out_specs=pl.BlockSpec((bi, _NCPAD), lambda i: (i, 0)),
        out_shape=jax.ShapeDtypeStruct((n, _NCPAD), jnp.float32),
        compiler_params=pltpu.CompilerParams(
            dimension_semantics=("parallel",)),
    )(adj, g, b4r)

    return out[:, :nclass]
